# baseline (speedup 1.0000x reference)
import functools
import math

import jax
import jax.numpy as jnp
from jax import lax
from jax.experimental import pallas as pl
from jax.experimental.pallas import tpu as pltpu
from jax.experimental.pallas import tpu_sc as plsc

N = 4096
NB = 2
K = 16
CH = [256, 512, 512, 512]
SCALE = math.atanh(0.5) / 0.02


def _ft_body(x_ref, w0_ref, b0_ref, w1_ref, b1_ref, o_ref):
    x = x_ref[...]
    h0 = jnp.dot(x[:, :8], w0_ref[...], preferred_element_type=jnp.float32)
    h1 = jnp.dot(x[:, 8:], w1_ref[...], preferred_element_type=jnp.float32)
    o_ref[:N, :] = h0 + b0_ref[...]
    o_ref[N:, :] = h1 + b1_ref[...]


def _feature_transform(x, w0, b0, w1, b1):
    return pl.pallas_call(
        _ft_body,
        out_shape=jax.ShapeDtypeStruct((2 * N, 256), jnp.float32),
    )(x, w0, b0[None, :], w1, b1[None, :])


def _hlr_body(x_ref, w_ref, o_ref):
    o_ref[...] = jnp.dot(x_ref[...], w_ref[...], preferred_element_type=jnp.float32)


def _hlr(x, w2):
    nt, ci = x.shape
    co2 = w2.shape[1]
    blk = 2048
    return pl.pallas_call(
        _hlr_body,
        grid=(nt // blk,),
        in_specs=[
            pl.BlockSpec((blk, ci), lambda i: (i, 0)),
            pl.BlockSpec((ci, co2), lambda i: (0, 0)),
        ],
        out_specs=pl.BlockSpec((blk, co2), lambda i: (i, 0)),
        out_shape=jax.ShapeDtypeStruct((nt, co2), jnp.float32),
    )(x, w2)


_BIG_F = 3e38
_BIG_I = 2**30


def _extract_topk(v, ix, k):
    vals, idxs = [], []
    for _ in range(k):
        m = jnp.min(v, axis=1, keepdims=True)
        am = jnp.min(jnp.where(v == m, ix, jnp.int32(_BIG_I)), axis=1,
                     keepdims=True)
        vals.append(m)
        idxs.append(am)
        v = jnp.where(ix == am, jnp.float32(_BIG_F), v)
    return vals, idxs


def _topk_body(x_ref, xt_ref, idx_ref, *, n, br, gens):
    g = pl.program_id(0)
    i = pl.program_id(1)
    x_rows = x_ref[0]
    xt = xt_ref[0]
    sq_rows = jnp.sum(x_rows * x_rows, axis=1, keepdims=True)
    sq_all = jnp.sum(xt * xt, axis=0, keepdims=True)
    s = jnp.dot(x_rows, xt, preferred_element_type=jnp.float32)
    col = lax.broadcasted_iota(jnp.int32, (br, n), 1)
    row = i * br + lax.broadcasted_iota(jnp.int32, (br, n), 0)
    d2 = sq_rows + sq_all - 2.0 * s
    d2 = jnp.where(col == row, d2 + 1e9, d2)

    arrays = [(d2, col, K)]
    for _ in range(gens):
        nxt = []
        for v, ix, k in arrays:
            w = v.shape[1] // 2
            lv, rv = v[:, :w], v[:, w:]
            li, ri = ix[:, :w], ix[:, w:]
            cond = lv <= rv
            mn = jnp.where(cond, lv, rv)
            mni = jnp.where(cond, li, ri)
            nxt.append((mn, mni, k))
            if k >= 2:
                mx = jnp.where(cond, rv, lv)
                mxi = jnp.where(cond, ri, li)
                nxt.append((mx, mxi, k // 2))
        arrays = nxt

    states = [[v, ix, k] for v, ix, k in arrays]
    cand_v = [[] for _ in states]
    cand_i = [[] for _ in states]
    for r in range(K):
        for a, st in enumerate(states):
            v, ix, k = st
            if r >= k:
                continue
            m = jnp.min(v, axis=1, keepdims=True)
            am = jnp.min(jnp.where(v == m, ix, jnp.int32(_BIG_I)), axis=1,
                         keepdims=True)
            cand_v[a].append(m)
            cand_i[a].append(am)
            st[0] = jnp.where(ix == am, jnp.float32(_BIG_F), v)
    cv = jnp.concatenate(sum(cand_v, []), axis=1)
    ci_ = jnp.concatenate(sum(cand_i, []), axis=1)
    _, final_i = _extract_topk(cv, ci_, K)
    idx_ref[0] = jnp.concatenate(final_i, axis=1) + g * n


def _topk(x, groups):
    nt, ci = x.shape
    n = nt // groups
    br = 256
    xg = x.reshape(groups, n, ci)
    xt = xg.transpose(0, 2, 1)
    body = functools.partial(_topk_body, n=n, br=br, gens=3)
    idx = pl.pallas_call(
        body,
        grid=(groups, n // br),
        in_specs=[
            pl.BlockSpec((1, br, ci), lambda g, i: (g, i, 0)),
            pl.BlockSpec((1, ci, n), lambda g, i: (g, 0, 0)),
        ],
        out_specs=pl.BlockSpec((1, br, K), lambda g, i: (g, i, 0)),
        out_shape=jax.ShapeDtypeStruct((groups, n, K), jnp.int32),
    )(xg, xt)
    return idx.reshape(nt, K)


def _attn_compute(gg, hl, av, br, co):
    g3 = gg.reshape(br, K, co)
    u = g3 + hl[:, None, :]
    v = jnp.where(u >= 0, u, 0.2 * u)
    e = jnp.sum(v.astype(jnp.bfloat16).astype(jnp.float32)
                * av.astype(jnp.bfloat16).astype(jnp.float32), axis=2)
    emax = jnp.max(e, axis=1, keepdims=True)
    ee = jnp.exp(e - emax)
    den = jnp.sum(ee, axis=1, keepdims=True)
    alpha = ee / den
    return jnp.sum(g3 * alpha[:, :, None], axis=1)


def _attn_body(g_ref, hl_ref, att_ref, b_ref, o_ref, *, co, br):
    av = att_ref[...].reshape(1, 1, co)
    o = _attn_compute(g_ref[...], hl_ref[...], av, br, co)
    o_ref[...] = o + b_ref[...]


def _attention(gath, hl, att, b):
    nt, co = hl.shape
    br = 256
    body = functools.partial(_attn_body, co=co, br=br)
    return pl.pallas_call(
        body,
        grid=(nt // br,),
        in_specs=[
            pl.BlockSpec((br * K, co), lambda i: (i, 0)),
            pl.BlockSpec((br, co), lambda i: (i, 0)),
            pl.BlockSpec((1, co), lambda i: (0, 0)),
            pl.BlockSpec((1, co), lambda i: (0, 0)),
        ],
        out_specs=pl.BlockSpec((br, co), lambda i: (i, 0)),
        out_shape=jax.ShapeDtypeStruct((nt, co), jnp.float32),
    )(gath, hl, att[None, :], b[None, :])


def _attn_body_g2(g_ref, hl_ref, att_ref, b_ref, o_ref, *, co, br):
    av = att_ref[...].reshape(1, 1, co)
    o0 = _attn_compute(g_ref[0], hl_ref[0], av, br, co)
    o1 = _attn_compute(g_ref[1], hl_ref[1], av, br, co)
    o_ref[...] = (o0 + o1) * 0.5 + b_ref[...]


def _attention_g2(gath, hl, att, b):
    co = hl.shape[1]
    n = hl.shape[0] // 2
    br = 256
    body = functools.partial(_attn_body_g2, co=co, br=br)
    return pl.pallas_call(
        body,
        grid=(n // br,),
        in_specs=[
            pl.BlockSpec((2, br * K, co), lambda i: (0, i, 0)),
            pl.BlockSpec((2, br, co), lambda i: (0, i, 0)),
            pl.BlockSpec((1, co), lambda i: (0, 0)),
            pl.BlockSpec((1, co), lambda i: (0, 0)),
        ],
        out_specs=pl.BlockSpec((br, co), lambda i: (i, 0)),
        out_shape=jax.ShapeDtypeStruct((n, co), jnp.float32),
    )(gath.reshape(2, n * K, co), hl.reshape(2, n, co), att[None, :], b[None, :])


def _mlp_body(x_ref, w1_ref, b1_ref, w2_ref, b2_ref, wg_ref, bg_ref, geod_ref,
              o_ref):
    h = jnp.dot(x_ref[...], w1_ref[...], preferred_element_type=jnp.float32)
    h = jnp.maximum(h + b1_ref[...], 0.0)
    h = jnp.dot(h, w2_ref[...], preferred_element_type=jnp.float32)
    h = jnp.maximum(h + b2_ref[...], 0.0)
    z = jnp.dot(h, wg_ref[...], preferred_element_type=jnp.float32) + bg_ref[...]
    o_ref[...] = jnp.tanh(z) * geod_ref[...] / SCALE


def _mlp(x, w1, b1, w2, b2, wg, bg, geod):
    nt = x.shape[0]
    wg_p = jnp.zeros((64, 128), jnp.float32).at[:, :3].set(wg)
    bg_p = jnp.zeros((128,), jnp.float32).at[:3].set(bg)
    out = pl.pallas_call(
        _mlp_body,
        out_shape=jax.ShapeDtypeStruct((nt, 128), jnp.float32),
    )(x, w1, b1[None, :], w2, b2[None, :], wg_p, bg_p[None, :], geod[:, None])
    return out[:, :3]


def _sc_gather(hr, idx):
    E = idx.shape[0]
    nt, co = hr.shape
    info = plsc.get_sparse_core_info()
    nw = info.num_cores * info.num_subcores
    epw = E // nw
    ch = 128
    nch = epw // ch
    mesh = plsc.VectorSubcoreMesh(core_axis_name="c", subcore_axis_name="s")

    @functools.partial(
        pl.kernel, mesh=mesh,
        out_type=jax.ShapeDtypeStruct((E, co), jnp.float32),
        scratch_types=[
            pltpu.VMEM((ch,), jnp.int32),
            pltpu.VMEM((ch, co), jnp.float32),
            pltpu.SemaphoreType.DMA,
        ],
    )
    def k(hr_hbm, idx_hbm, out_hbm, idx_v, rows_v, sem):
        wid = lax.axis_index("s") * info.num_cores + lax.axis_index("c")
        base = wid * epw

        def body(c, carry):
            off = base + c * ch
            pltpu.sync_copy(idx_hbm.at[pl.ds(off, ch)], idx_v)
            pltpu.async_copy(hr_hbm.at[idx_v], rows_v, sem).wait()
            pltpu.sync_copy(rows_v, out_hbm.at[pl.ds(off, ch)])
            return carry

        lax.fori_loop(0, nch, body, 0)

    return k(hr, idx)


def _gat_layer(x, params, prefix, groups):
    nt, ci = x.shape
    wl = params[prefix + '_Wl']
    wr = params[prefix + '_Wr']
    co = wl.shape[1]
    h = _hlr(x, jnp.concatenate([wl, wr], axis=1))
    hl, hr = h[:, :co], h[:, co:]
    idx = _topk(x, groups)
    gath = _sc_gather(hr, idx.reshape(-1))
    if groups == 1:
        return _attention(gath, hl, params[prefix + '_att'], params[prefix + '_b'])
    return _attention_g2(gath, hl, params[prefix + '_att'], params[prefix + '_b'])


def kernel(x, params):
    xb = _feature_transform(x, params['ft_W0'], params['ft_b0'],
                            params['ft_W1'], params['ft_b1'])
    y = _gat_layer(xb, params, 'l1', NB)
    outs = [y]
    for i in range(2, len(CH) + 1):
        outs.append(_gat_layer(outs[-1], params, 'l' + str(i), 1))
    out = jnp.concatenate(outs, axis=1)
    return _mlp(out, params['m1_W'], params['m1_b'], params['m2_W'],
                params['m2_b'], params['gl_W'], params['gl_b'], params['geod'])


# speedup vs baseline: 1.0795x; 1.0795x over previous
import functools
import math

import jax
import jax.numpy as jnp
from jax import lax
from jax.experimental import pallas as pl
from jax.experimental.pallas import tpu as pltpu
from jax.experimental.pallas import tpu_sc as plsc

N = 4096
NB = 2
K = 16
CH = [256, 512, 512, 512]
SCALE = math.atanh(0.5) / 0.02


def _ft_body(x_ref, w0_ref, b0_ref, w1_ref, b1_ref, o_ref):
    x = x_ref[...]
    h0 = jnp.dot(x[:, :8], w0_ref[...], preferred_element_type=jnp.float32)
    h1 = jnp.dot(x[:, 8:], w1_ref[...], preferred_element_type=jnp.float32)
    o_ref[:N, :] = h0 + b0_ref[...]
    o_ref[N:, :] = h1 + b1_ref[...]


def _feature_transform(x, w0, b0, w1, b1):
    return pl.pallas_call(
        _ft_body,
        out_shape=jax.ShapeDtypeStruct((2 * N, 256), jnp.float32),
    )(x, w0, b0[None, :], w1, b1[None, :])


def _hlr_body(x_ref, w_ref, o_ref):
    o_ref[...] = jnp.dot(x_ref[...], w_ref[...], preferred_element_type=jnp.float32)


def _hlr(x, w2):
    nt, ci = x.shape
    co2 = w2.shape[1]
    blk = 2048
    return pl.pallas_call(
        _hlr_body,
        grid=(nt // blk,),
        in_specs=[
            pl.BlockSpec((blk, ci), lambda i: (i, 0)),
            pl.BlockSpec((ci, co2), lambda i: (0, 0)),
        ],
        out_specs=pl.BlockSpec((blk, co2), lambda i: (i, 0)),
        out_shape=jax.ShapeDtypeStruct((nt, co2), jnp.float32),
    )(x, w2)


_BIG_F = 3e38
_BIG_I = 2**30


def _extract_topk(v, ix, k):
    vals, idxs = [], []
    for _ in range(k):
        m = jnp.min(v, axis=1, keepdims=True)
        am = jnp.min(jnp.where(v == m, ix, jnp.int32(_BIG_I)), axis=1,
                     keepdims=True)
        vals.append(m)
        idxs.append(am)
        v = jnp.where(ix == am, jnp.float32(_BIG_F), v)
    return vals, idxs


def _topk_body(x_ref, xt_ref, idx_ref, *, n, br, gens):
    g = pl.program_id(0)
    i = pl.program_id(1)
    x_rows = x_ref[0]
    xt = xt_ref[0]
    sq_rows = jnp.sum(x_rows * x_rows, axis=1, keepdims=True)
    sq_all = jnp.sum(xt * xt, axis=0, keepdims=True)
    s = jnp.dot(x_rows, xt, preferred_element_type=jnp.float32)
    col = lax.broadcasted_iota(jnp.int32, (br, n), 1)
    row = i * br + lax.broadcasted_iota(jnp.int32, (br, n), 0)
    d2 = sq_rows + sq_all - 2.0 * s
    d2 = jnp.where(col == row, d2 + 1e9, d2)

    arrays = [(d2, col, K)]
    for _ in range(gens):
        nxt = []
        for v, ix, k in arrays:
            w = v.shape[1] // 2
            lv, rv = v[:, :w], v[:, w:]
            li, ri = ix[:, :w], ix[:, w:]
            cond = lv <= rv
            mn = jnp.where(cond, lv, rv)
            mni = jnp.where(cond, li, ri)
            nxt.append((mn, mni, k))
            if k >= 2:
                mx = jnp.where(cond, rv, lv)
                mxi = jnp.where(cond, ri, li)
                nxt.append((mx, mxi, k // 2))
        arrays = nxt

    cand_v, cand_i = [], []
    for v, ix, k in arrays:
        vs, ixs = _extract_topk(v, ix, k)
        cand_v += vs
        cand_i += ixs
    cv = jnp.concatenate(cand_v, axis=1)
    ci_ = jnp.concatenate(cand_i, axis=1)
    _, final_i = _extract_topk(cv, ci_, K)
    idx_ref[0] = jnp.concatenate(final_i, axis=1) + g * n


def _topk(x, groups):
    nt, ci = x.shape
    n = nt // groups
    br = 512
    xg = x.reshape(groups, n, ci)
    xt = xg.transpose(0, 2, 1)
    body = functools.partial(_topk_body, n=n, br=br, gens=2)
    idx = pl.pallas_call(
        body,
        grid=(groups, n // br),
        in_specs=[
            pl.BlockSpec((1, br, ci), lambda g, i: (g, i, 0)),
            pl.BlockSpec((1, ci, n), lambda g, i: (g, 0, 0)),
        ],
        out_specs=pl.BlockSpec((1, br, K), lambda g, i: (g, i, 0)),
        out_shape=jax.ShapeDtypeStruct((groups, n, K), jnp.int32),
    )(xg, xt)
    return idx.reshape(nt, K)


def _attn_compute(gg, hl, av, br, co):
    g3 = gg.reshape(br, K, co)
    u = g3 + hl[:, None, :]
    v = jnp.where(u >= 0, u, 0.2 * u)
    e = jnp.sum(v.astype(jnp.bfloat16).astype(jnp.float32)
                * av.astype(jnp.bfloat16).astype(jnp.float32), axis=2)
    emax = jnp.max(e, axis=1, keepdims=True)
    ee = jnp.exp(e - emax)
    den = jnp.sum(ee, axis=1, keepdims=True)
    alpha = ee / den
    return jnp.sum(g3 * alpha[:, :, None], axis=1)


def _attn_body(g_ref, hl_ref, att_ref, b_ref, o_ref, *, co, br):
    av = att_ref[...].reshape(1, 1, co)
    o = _attn_compute(g_ref[...], hl_ref[...], av, br, co)
    o_ref[...] = o + b_ref[...]


def _attention(gath, hl, att, b):
    nt, co = hl.shape
    br = 512
    body = functools.partial(_attn_body, co=co, br=br)
    return pl.pallas_call(
        body,
        grid=(nt // br,),
        in_specs=[
            pl.BlockSpec((br * K, co), lambda i: (i, 0)),
            pl.BlockSpec((br, co), lambda i: (i, 0)),
            pl.BlockSpec((1, co), lambda i: (0, 0)),
            pl.BlockSpec((1, co), lambda i: (0, 0)),
        ],
        out_specs=pl.BlockSpec((br, co), lambda i: (i, 0)),
        out_shape=jax.ShapeDtypeStruct((nt, co), jnp.float32),
    )(gath, hl, att[None, :], b[None, :])


def _attn_body_g2(g_ref, hl_ref, att_ref, b_ref, o_ref, *, co, br):
    av = att_ref[...].reshape(1, 1, co)
    o0 = _attn_compute(g_ref[0], hl_ref[0], av, br, co)
    o1 = _attn_compute(g_ref[1], hl_ref[1], av, br, co)
    o_ref[...] = (o0 + o1) * 0.5 + b_ref[...]


def _attention_g2(gath, hl, att, b):
    co = hl.shape[1]
    n = hl.shape[0] // 2
    br = 512
    body = functools.partial(_attn_body_g2, co=co, br=br)
    return pl.pallas_call(
        body,
        grid=(n // br,),
        in_specs=[
            pl.BlockSpec((2, br * K, co), lambda i: (0, i, 0)),
            pl.BlockSpec((2, br, co), lambda i: (0, i, 0)),
            pl.BlockSpec((1, co), lambda i: (0, 0)),
            pl.BlockSpec((1, co), lambda i: (0, 0)),
        ],
        out_specs=pl.BlockSpec((br, co), lambda i: (i, 0)),
        out_shape=jax.ShapeDtypeStruct((n, co), jnp.float32),
    )(gath.reshape(2, n * K, co), hl.reshape(2, n, co), att[None, :], b[None, :])


def _mlp_body(x_ref, w1_ref, b1_ref, w2_ref, b2_ref, wg_ref, bg_ref, geod_ref,
              o_ref):
    h = jnp.dot(x_ref[...], w1_ref[...], preferred_element_type=jnp.float32)
    h = jnp.maximum(h + b1_ref[...], 0.0)
    h = jnp.dot(h, w2_ref[...], preferred_element_type=jnp.float32)
    h = jnp.maximum(h + b2_ref[...], 0.0)
    z = jnp.dot(h, wg_ref[...], preferred_element_type=jnp.float32) + bg_ref[...]
    o_ref[...] = jnp.tanh(z) * geod_ref[...] / SCALE


def _mlp(x, w1, b1, w2, b2, wg, bg, geod):
    nt = x.shape[0]
    wg_p = jnp.zeros((64, 128), jnp.float32).at[:, :3].set(wg)
    bg_p = jnp.zeros((128,), jnp.float32).at[:3].set(bg)
    out = pl.pallas_call(
        _mlp_body,
        out_shape=jax.ShapeDtypeStruct((nt, 128), jnp.float32),
    )(x, w1, b1[None, :], w2, b2[None, :], wg_p, bg_p[None, :], geod[:, None])
    return out[:, :3]


def _sc_gather(hr, idx):
    E = idx.shape[0]
    nt, co = hr.shape
    info = plsc.get_sparse_core_info()
    nw = info.num_cores * info.num_subcores
    epw = E // nw
    ch = 128
    nch = epw // ch
    mesh = plsc.VectorSubcoreMesh(core_axis_name="c", subcore_axis_name="s")

    @functools.partial(
        pl.kernel, mesh=mesh,
        out_type=jax.ShapeDtypeStruct((E, co), jnp.float32),
        scratch_types=[
            pltpu.VMEM((ch,), jnp.int32),
            pltpu.VMEM((ch, co), jnp.float32),
            pltpu.SemaphoreType.DMA,
        ],
    )
    def k(hr_hbm, idx_hbm, out_hbm, idx_v, rows_v, sem):
        wid = lax.axis_index("s") * info.num_cores + lax.axis_index("c")
        base = wid * epw

        def body(c, carry):
            off = base + c * ch
            pltpu.sync_copy(idx_hbm.at[pl.ds(off, ch)], idx_v)
            pltpu.async_copy(hr_hbm.at[idx_v], rows_v, sem).wait()
            pltpu.sync_copy(rows_v, out_hbm.at[pl.ds(off, ch)])
            return carry

        lax.fori_loop(0, nch, body, 0)

    return k(hr, idx)


def _gat_layer(x, params, prefix, groups):
    nt, ci = x.shape
    wl = params[prefix + '_Wl']
    wr = params[prefix + '_Wr']
    co = wl.shape[1]
    h = _hlr(x, jnp.concatenate([wl, wr], axis=1))
    hl, hr = h[:, :co], h[:, co:]
    idx = _topk(x, groups)
    gath = _sc_gather(hr, idx.reshape(-1))
    if groups == 1:
        return _attention(gath, hl, params[prefix + '_att'], params[prefix + '_b'])
    return _attention_g2(gath, hl, params[prefix + '_att'], params[prefix + '_b'])


def kernel(x, params):
    xb = _feature_transform(x, params['ft_W0'], params['ft_b0'],
                            params['ft_W1'], params['ft_b1'])
    y = _gat_layer(xb, params, 'l1', NB)
    outs = [y]
    for i in range(2, len(CH) + 1):
        outs.append(_gat_layer(outs[-1], params, 'l' + str(i), 1))
    out = jnp.concatenate(outs, axis=1)
    return _mlp(out, params['m1_W'], params['m1_b'], params['m2_W'],
                params['m2_b'], params['gl_W'], params['gl_b'], params['geod'])


# speedup vs baseline: 1.0976x; 1.0167x over previous
import functools
import math

import jax
import jax.numpy as jnp
from jax import lax
from jax.experimental import pallas as pl
from jax.experimental.pallas import tpu as pltpu
from jax.experimental.pallas import tpu_sc as plsc

N = 4096
NB = 2
K = 16
CH = [256, 512, 512, 512]
SCALE = math.atanh(0.5) / 0.02


def _ft_body(x_ref, w0_ref, b0_ref, w1_ref, b1_ref, o_ref):
    x = x_ref[...]
    h0 = jnp.dot(x[:, :8], w0_ref[...], preferred_element_type=jnp.float32)
    h1 = jnp.dot(x[:, 8:], w1_ref[...], preferred_element_type=jnp.float32)
    o_ref[:N, :] = h0 + b0_ref[...]
    o_ref[N:, :] = h1 + b1_ref[...]


def _feature_transform(x, w0, b0, w1, b1):
    return pl.pallas_call(
        _ft_body,
        out_shape=jax.ShapeDtypeStruct((2 * N, 256), jnp.float32),
    )(x, w0, b0[None, :], w1, b1[None, :])


def _hlr_body(x_ref, w_ref, o_ref):
    o_ref[...] = jnp.dot(x_ref[...], w_ref[...], preferred_element_type=jnp.float32)


def _hlr(x, w2):
    nt, ci = x.shape
    co2 = w2.shape[1]
    blk = 2048
    return pl.pallas_call(
        _hlr_body,
        grid=(nt // blk,),
        in_specs=[
            pl.BlockSpec((blk, ci), lambda i: (i, 0)),
            pl.BlockSpec((ci, co2), lambda i: (0, 0)),
        ],
        out_specs=pl.BlockSpec((blk, co2), lambda i: (i, 0)),
        out_shape=jax.ShapeDtypeStruct((nt, co2), jnp.float32),
    )(x, w2)


_BIG_F = 3e38
_BIG_I = 2**30


def _extract_topk(v, ix, k):
    vals, idxs = [], []
    for _ in range(k):
        m = jnp.min(v, axis=1, keepdims=True)
        am = jnp.min(jnp.where(v == m, ix, jnp.int32(_BIG_I)), axis=1,
                     keepdims=True)
        vals.append(m)
        idxs.append(am)
        v = jnp.where(ix == am, jnp.float32(_BIG_F), v)
    return vals, idxs


def _topk_body(x_ref, xt_ref, idx_ref, *, n, br, gens):
    g = pl.program_id(0)
    i = pl.program_id(1)
    x_rows = x_ref[0]
    xt = xt_ref[0]
    sq_rows = jnp.sum(x_rows * x_rows, axis=1, keepdims=True)
    sq_all = jnp.sum(xt * xt, axis=0, keepdims=True)
    s = jnp.dot(x_rows, xt, preferred_element_type=jnp.float32)
    col = lax.broadcasted_iota(jnp.int32, (br, n), 1)
    row = i * br + lax.broadcasted_iota(jnp.int32, (br, n), 0)
    d2 = sq_rows + sq_all - 2.0 * s
    d2 = jnp.where(col == row, d2 + 1e9, d2)

    arrays = [(d2, col, K)]
    for _ in range(gens):
        nxt = []
        for v, ix, k in arrays:
            w = v.shape[1] // 2
            lv, rv = v[:, :w], v[:, w:]
            li, ri = ix[:, :w], ix[:, w:]
            cond = lv <= rv
            mn = jnp.where(cond, lv, rv)
            mni = jnp.where(cond, li, ri)
            nxt.append((mn, mni, k))
            if k >= 2:
                mx = jnp.where(cond, rv, lv)
                mxi = jnp.where(cond, ri, li)
                nxt.append((mx, mxi, k // 2))
        arrays = nxt

    cand_v, cand_i = [], []
    for v, ix, k in arrays:
        vs, ixs = _extract_topk(v, ix, k)
        cand_v += vs
        cand_i += ixs
    cv = jnp.concatenate(cand_v, axis=1)
    ci_ = jnp.concatenate(cand_i, axis=1)
    _, final_i = _extract_topk(cv, ci_, K)
    idx_ref[0] = jnp.concatenate(final_i, axis=1) + g * n


def _topk(x, groups):
    nt, ci = x.shape
    n = nt // groups
    br = 512
    xg = x.reshape(groups, n, ci)
    xt = xg.transpose(0, 2, 1)
    body = functools.partial(_topk_body, n=n, br=br, gens=2)
    idx = pl.pallas_call(
        body,
        grid=(groups, n // br),
        in_specs=[
            pl.BlockSpec((1, br, ci), lambda g, i: (g, i, 0)),
            pl.BlockSpec((1, ci, n), lambda g, i: (g, 0, 0)),
        ],
        out_specs=pl.BlockSpec((1, br, K), lambda g, i: (g, i, 0)),
        out_shape=jax.ShapeDtypeStruct((groups, n, K), jnp.int32),
    )(xg, xt)
    return idx.reshape(nt, K)


def _attn_compute(gg, hl, av, br, co):
    g3 = gg.reshape(br, K, co)
    u = g3 + hl[:, None, :]
    v = jnp.where(u >= 0, u, 0.2 * u)
    e = jnp.sum(v.astype(jnp.bfloat16).astype(jnp.float32)
                * av.astype(jnp.bfloat16).astype(jnp.float32), axis=2)
    emax = jnp.max(e, axis=1, keepdims=True)
    ee = jnp.exp(e - emax)
    den = jnp.sum(ee, axis=1, keepdims=True)
    alpha = ee / den
    return jnp.sum(g3 * alpha[:, :, None], axis=1)


def _attn_body(g_ref, hl_ref, att_ref, b_ref, o_ref, *, co, br):
    av = att_ref[...].reshape(1, 1, co)
    o = _attn_compute(g_ref[...], hl_ref[...], av, br, co)
    o_ref[...] = o + b_ref[...]


def _attention(gath, hl, att, b):
    nt, co = hl.shape
    br = 512
    body = functools.partial(_attn_body, co=co, br=br)
    return pl.pallas_call(
        body,
        grid=(nt // br,),
        in_specs=[
            pl.BlockSpec((br * K, co), lambda i: (i, 0)),
            pl.BlockSpec((br, co), lambda i: (i, 0)),
            pl.BlockSpec((1, co), lambda i: (0, 0)),
            pl.BlockSpec((1, co), lambda i: (0, 0)),
        ],
        out_specs=pl.BlockSpec((br, co), lambda i: (i, 0)),
        out_shape=jax.ShapeDtypeStruct((nt, co), jnp.float32),
    )(gath, hl, att[None, :], b[None, :])


def _attn_body_g2(g_ref, hl_ref, att_ref, b_ref, o_ref, *, co, br):
    av = att_ref[...].reshape(1, 1, co)
    o0 = _attn_compute(g_ref[0], hl_ref[0], av, br, co)
    o1 = _attn_compute(g_ref[1], hl_ref[1], av, br, co)
    o_ref[...] = (o0 + o1) * 0.5 + b_ref[...]


def _attention_g2(gath, hl, att, b):
    co = hl.shape[1]
    n = hl.shape[0] // 2
    br = 512
    body = functools.partial(_attn_body_g2, co=co, br=br)
    return pl.pallas_call(
        body,
        grid=(n // br,),
        in_specs=[
            pl.BlockSpec((2, br * K, co), lambda i: (0, i, 0)),
            pl.BlockSpec((2, br, co), lambda i: (0, i, 0)),
            pl.BlockSpec((1, co), lambda i: (0, 0)),
            pl.BlockSpec((1, co), lambda i: (0, 0)),
        ],
        out_specs=pl.BlockSpec((br, co), lambda i: (i, 0)),
        out_shape=jax.ShapeDtypeStruct((n, co), jnp.float32),
    )(gath.reshape(2, n * K, co), hl.reshape(2, n, co), att[None, :], b[None, :])


def _mlp_body(x_ref, w1_ref, b1_ref, w2_ref, b2_ref, wg_ref, bg_ref, geod_ref,
              o_ref):
    h = jnp.dot(x_ref[...], w1_ref[...], preferred_element_type=jnp.float32)
    h = jnp.maximum(h + b1_ref[...], 0.0)
    h = jnp.dot(h, w2_ref[...], preferred_element_type=jnp.float32)
    h = jnp.maximum(h + b2_ref[...], 0.0)
    z = jnp.dot(h, wg_ref[...], preferred_element_type=jnp.float32) + bg_ref[...]
    o_ref[...] = jnp.tanh(z) * geod_ref[...] / SCALE


def _mlp(x, w1, b1, w2, b2, wg, bg, geod):
    nt = x.shape[0]
    wg_p = jnp.zeros((64, 128), jnp.float32).at[:, :3].set(wg)
    bg_p = jnp.zeros((128,), jnp.float32).at[:3].set(bg)
    out = pl.pallas_call(
        _mlp_body,
        out_shape=jax.ShapeDtypeStruct((nt, 128), jnp.float32),
    )(x, w1, b1[None, :], w2, b2[None, :], wg_p, bg_p[None, :], geod[:, None])
    return out[:, :3]


def _sc_gather(hr, idx):
    E = idx.shape[0]
    nt, co = hr.shape
    info = plsc.get_sparse_core_info()
    nw = info.num_cores * info.num_subcores
    epw = E // nw
    ch = max(8, 131072 // (co * 4))
    nch = epw // ch
    mesh = plsc.VectorSubcoreMesh(core_axis_name="c", subcore_axis_name="s")

    @functools.partial(
        pl.kernel, mesh=mesh,
        out_type=jax.ShapeDtypeStruct((E, co), jnp.float32),
        scratch_types=[
            pltpu.VMEM((epw,), jnp.int32),
            pltpu.VMEM((ch, co), jnp.float32),
            pltpu.VMEM((ch, co), jnp.float32),
            pltpu.SemaphoreType.DMA,
            pltpu.SemaphoreType.DMA,
            pltpu.SemaphoreType.DMA,
            pltpu.SemaphoreType.DMA,
        ],
    )
    def k(hr_hbm, idx_hbm, out_hbm, idx_v, rows0, rows1, sg0, sg1, sw0, sw1):
        wid = lax.axis_index("s") * info.num_cores + lax.axis_index("c")
        base = wid * epw
        pltpu.sync_copy(idx_hbm.at[pl.ds(base, epw)], idx_v)
        rows = (rows0, rows1)
        sg = (sg0, sg1)
        sw = (sw0, sw1)
        gathers = [None, None]
        writes = [None, None]
        for c in range(nch):
            b = c % 2
            if writes[b] is not None:
                writes[b].wait()
                writes[b] = None
            gathers[b] = pltpu.async_copy(
                hr_hbm.at[idx_v.at[pl.ds(c * ch, ch)]], rows[b], sg[b])
            gathers[b].wait()
            writes[b] = pltpu.async_copy(
                rows[b], out_hbm.at[pl.ds(base + c * ch, ch)], sw[b])
        for b in range(2):
            if writes[b] is not None:
                writes[b].wait()

    return k(hr, idx)


def _gat_layer(x, params, prefix, groups):
    nt, ci = x.shape
    wl = params[prefix + '_Wl']
    wr = params[prefix + '_Wr']
    co = wl.shape[1]
    h = _hlr(x, jnp.concatenate([wl, wr], axis=1))
    hl, hr = h[:, :co], h[:, co:]
    idx = _topk(x, groups)
    gath = _sc_gather(hr, idx.reshape(-1))
    if groups == 1:
        return _attention(gath, hl, params[prefix + '_att'], params[prefix + '_b'])
    return _attention_g2(gath, hl, params[prefix + '_att'], params[prefix + '_b'])


def kernel(x, params):
    xb = _feature_transform(x, params['ft_W0'], params['ft_b0'],
                            params['ft_W1'], params['ft_b1'])
    y = _gat_layer(xb, params, 'l1', NB)
    outs = [y]
    for i in range(2, len(CH) + 1):
        outs.append(_gat_layer(outs[-1], params, 'l' + str(i), 1))
    out = jnp.concatenate(outs, axis=1)
    return _mlp(out, params['m1_W'], params['m1_b'], params['m2_W'],
                params['m2_b'], params['gl_W'], params['gl_b'], params['geod'])


# speedup vs baseline: 1.1238x; 1.0238x over previous
import functools
import math

import jax
import jax.numpy as jnp
from jax import lax
from jax.experimental import pallas as pl
from jax.experimental.pallas import tpu as pltpu
from jax.experimental.pallas import tpu_sc as plsc

N = 4096
NB = 2
K = 16
CH = [256, 512, 512, 512]
SCALE = math.atanh(0.5) / 0.02


def _ft_body(x_ref, w0_ref, b0_ref, w1_ref, b1_ref, o_ref):
    x = x_ref[...]
    h0 = jnp.dot(x[:, :8], w0_ref[...], preferred_element_type=jnp.float32)
    h1 = jnp.dot(x[:, 8:], w1_ref[...], preferred_element_type=jnp.float32)
    o_ref[:N, :] = h0 + b0_ref[...]
    o_ref[N:, :] = h1 + b1_ref[...]


def _feature_transform(x, w0, b0, w1, b1):
    return pl.pallas_call(
        _ft_body,
        out_shape=jax.ShapeDtypeStruct((2 * N, 256), jnp.float32),
    )(x, w0, b0[None, :], w1, b1[None, :])


def _hlr_body(x_ref, w_ref, o_ref):
    o_ref[...] = jnp.dot(x_ref[...], w_ref[...], preferred_element_type=jnp.float32)


def _hlr(x, w2):
    nt, ci = x.shape
    co2 = w2.shape[1]
    blk = 2048
    return pl.pallas_call(
        _hlr_body,
        grid=(nt // blk,),
        in_specs=[
            pl.BlockSpec((blk, ci), lambda i: (i, 0)),
            pl.BlockSpec((ci, co2), lambda i: (0, 0)),
        ],
        out_specs=pl.BlockSpec((blk, co2), lambda i: (i, 0)),
        out_shape=jax.ShapeDtypeStruct((nt, co2), jnp.float32),
    )(x, w2)


_BIG_F = 3e38
_BIG_I = 2**30


def _extract_topk(v, ix, k):
    vals, idxs = [], []
    for _ in range(k):
        m = jnp.min(v, axis=1, keepdims=True)
        am = jnp.min(jnp.where(v == m, ix, jnp.int32(_BIG_I)), axis=1,
                     keepdims=True)
        vals.append(m)
        idxs.append(am)
        v = jnp.where(ix == am, jnp.float32(_BIG_F), v)
    return vals, idxs


def _topk_body(x_ref, xt_ref, idx_ref, *, n, br, gens):
    g = pl.program_id(0)
    i = pl.program_id(1)
    x_rows = x_ref[0]
    xt = xt_ref[0]
    sq_rows = jnp.sum(x_rows * x_rows, axis=1, keepdims=True)
    sq_all = jnp.sum(xt * xt, axis=0, keepdims=True)
    s = jnp.dot(x_rows, xt, preferred_element_type=jnp.float32)
    col = lax.broadcasted_iota(jnp.int32, (br, n), 1)
    row = i * br + lax.broadcasted_iota(jnp.int32, (br, n), 0)
    d2 = sq_rows + sq_all - 2.0 * s
    d2 = jnp.where(col == row, d2 + 1e9, d2)

    arrays = [(d2, col, K)]
    for _ in range(gens):
        nxt = []
        for v, ix, k in arrays:
            w = v.shape[1] // 2
            lv, rv = v[:, :w], v[:, w:]
            li, ri = ix[:, :w], ix[:, w:]
            cond = lv <= rv
            mn = jnp.where(cond, lv, rv)
            mni = jnp.where(cond, li, ri)
            nxt.append((mn, mni, k))
            if k >= 2:
                mx = jnp.where(cond, rv, lv)
                mxi = jnp.where(cond, ri, li)
                nxt.append((mx, mxi, k // 2))
        arrays = nxt

    cand_v, cand_i = [], []
    for v, ix, k in arrays:
        vs, ixs = _extract_topk(v, ix, k)
        cand_v += vs
        cand_i += ixs
    cv = jnp.concatenate(cand_v, axis=1)
    ci_ = jnp.concatenate(cand_i, axis=1)
    _, final_i = _extract_topk(cv, ci_, K)
    idx_ref[0] = jnp.concatenate(final_i, axis=1) + g * n


def _topk(x, groups):
    nt, ci = x.shape
    n = nt // groups
    br = 512
    xg = x.reshape(groups, n, ci)
    xt = xg.transpose(0, 2, 1)
    body = functools.partial(_topk_body, n=n, br=br, gens=2)
    idx = pl.pallas_call(
        body,
        grid=(groups, n // br),
        in_specs=[
            pl.BlockSpec((1, br, ci), lambda g, i: (g, i, 0)),
            pl.BlockSpec((1, ci, n), lambda g, i: (g, 0, 0)),
        ],
        out_specs=pl.BlockSpec((1, br, K), lambda g, i: (g, i, 0)),
        out_shape=jax.ShapeDtypeStruct((groups, n, K), jnp.int32),
    )(xg, xt)
    return idx.reshape(nt, K)


def _attn_compute(gg, hl, av, br, co):
    g3 = gg.reshape(br, K, co)
    u = g3 + hl[:, None, :]
    v = jnp.where(u >= 0, u, 0.2 * u)
    e = jnp.sum(v.astype(jnp.bfloat16).astype(jnp.float32)
                * av.astype(jnp.bfloat16).astype(jnp.float32), axis=2)
    emax = jnp.max(e, axis=1, keepdims=True)
    ee = jnp.exp(e - emax)
    den = jnp.sum(ee, axis=1, keepdims=True)
    alpha = ee / den
    return jnp.sum(g3 * alpha[:, :, None], axis=1)


def _attn_body(g_ref, hl_ref, att_ref, b_ref, o_ref, *, co, br):
    av = att_ref[...].reshape(1, 1, co)
    o = _attn_compute(g_ref[...], hl_ref[...], av, br, co)
    o_ref[...] = o + b_ref[...]


def _attention(gath, hl, att, b):
    nt, co = hl.shape
    br = 512
    body = functools.partial(_attn_body, co=co, br=br)
    return pl.pallas_call(
        body,
        grid=(nt // br,),
        in_specs=[
            pl.BlockSpec((br * K, co), lambda i: (i, 0)),
            pl.BlockSpec((br, co), lambda i: (i, 0)),
            pl.BlockSpec((1, co), lambda i: (0, 0)),
            pl.BlockSpec((1, co), lambda i: (0, 0)),
        ],
        out_specs=pl.BlockSpec((br, co), lambda i: (i, 0)),
        out_shape=jax.ShapeDtypeStruct((nt, co), jnp.float32),
    )(gath, hl, att[None, :], b[None, :])


def _attn_body_g2(g_ref, hl_ref, att_ref, b_ref, o_ref, *, co, br):
    av = att_ref[...].reshape(1, 1, co)
    o0 = _attn_compute(g_ref[0], hl_ref[0], av, br, co)
    o1 = _attn_compute(g_ref[1], hl_ref[1], av, br, co)
    o_ref[...] = (o0 + o1) * 0.5 + b_ref[...]


def _attention_g2(gath, hl, att, b):
    co = hl.shape[1]
    n = hl.shape[0] // 2
    br = 512
    body = functools.partial(_attn_body_g2, co=co, br=br)
    return pl.pallas_call(
        body,
        grid=(n // br,),
        in_specs=[
            pl.BlockSpec((2, br * K, co), lambda i: (0, i, 0)),
            pl.BlockSpec((2, br, co), lambda i: (0, i, 0)),
            pl.BlockSpec((1, co), lambda i: (0, 0)),
            pl.BlockSpec((1, co), lambda i: (0, 0)),
        ],
        out_specs=pl.BlockSpec((br, co), lambda i: (i, 0)),
        out_shape=jax.ShapeDtypeStruct((n, co), jnp.float32),
    )(gath.reshape(2, n * K, co), hl.reshape(2, n, co), att[None, :], b[None, :])


def _mlp_body(x_ref, w1_ref, b1_ref, w2_ref, b2_ref, wg_ref, bg_ref, geod_ref,
              o_ref):
    h = jnp.dot(x_ref[...], w1_ref[...], preferred_element_type=jnp.float32)
    h = jnp.maximum(h + b1_ref[...], 0.0)
    h = jnp.dot(h, w2_ref[...], preferred_element_type=jnp.float32)
    h = jnp.maximum(h + b2_ref[...], 0.0)
    z = jnp.dot(h, wg_ref[...], preferred_element_type=jnp.float32) + bg_ref[...]
    o_ref[...] = jnp.tanh(z) * geod_ref[...] / SCALE


def _mlp(x, w1, b1, w2, b2, wg, bg, geod):
    nt = x.shape[0]
    wg_p = jnp.zeros((64, 128), jnp.float32).at[:, :3].set(wg)
    bg_p = jnp.zeros((128,), jnp.float32).at[:3].set(bg)
    out = pl.pallas_call(
        _mlp_body,
        out_shape=jax.ShapeDtypeStruct((nt, 128), jnp.float32),
    )(x, w1, b1[None, :], w2, b2[None, :], wg_p, bg_p[None, :], geod[:, None])
    return out[:, :3]


def _sc_gather(hr, idx):
    E = idx.shape[0]
    nt, co = hr.shape
    info = plsc.get_sparse_core_info()
    nw = info.num_cores * info.num_subcores
    epw = E // nw
    ch = max(8, 131072 // (co * 4))
    nch = epw // ch
    mesh = plsc.VectorSubcoreMesh(core_axis_name="c", subcore_axis_name="s")

    @functools.partial(
        pl.kernel, mesh=mesh,
        out_type=jax.ShapeDtypeStruct((E, co), jnp.float32),
        scratch_types=[
            pltpu.VMEM((epw,), jnp.int32),
            pltpu.VMEM((ch, co), jnp.float32),
            pltpu.VMEM((ch, co), jnp.float32),
            pltpu.SemaphoreType.DMA,
            pltpu.SemaphoreType.DMA,
            pltpu.SemaphoreType.DMA,
            pltpu.SemaphoreType.DMA,
        ],
    )
    def k(hr_hbm, idx_hbm, out_hbm, idx_v, rows0, rows1, sg0, sg1, sw0, sw1):
        wid = lax.axis_index("s") * info.num_cores + lax.axis_index("c")
        base = wid * epw
        pltpu.sync_copy(idx_hbm.at[pl.ds(base, epw)], idx_v)
        rows = (rows0, rows1)
        sg = (sg0, sg1)
        sw = (sw0, sw1)
        gathers = [None, None]
        writes = [None, None]
        for c in range(nch):
            b = c % 2
            if writes[b] is not None:
                writes[b].wait()
                writes[b] = None
            gathers[b] = pltpu.async_copy(
                hr_hbm.at[idx_v.at[pl.ds(c * ch, ch)]], rows[b], sg[b])
            gathers[b].wait()
            writes[b] = pltpu.async_copy(
                rows[b], out_hbm.at[pl.ds(base + c * ch, ch)], sw[b])
        for b in range(2):
            if writes[b] is not None:
                writes[b].wait()

    return k(hr, idx)


def _gat_layer(x, params, prefix, groups):
    nt, ci = x.shape
    wl = params[prefix + '_Wl']
    wr = params[prefix + '_Wr']
    co = wl.shape[1]
    h = _hlr(x, jnp.concatenate([wl, wr], axis=1))
    hl, hr = h[:, :co], h[:, co:]
    idx = _topk(x, groups)
    att = params[prefix + '_att']
    b = params[prefix + '_b']
    if groups == 1:
        half = nt // 2
        gath_a = _sc_gather(hr, idx[:half].reshape(-1))
        gath_b = _sc_gather(hr, idx[half:].reshape(-1))
        out_a = _attention(gath_a, hl[:half], att, b)
        out_b = _attention(gath_b, hl[half:], att, b)
        return jnp.concatenate([out_a, out_b], axis=0)
    gath = _sc_gather(hr, idx.reshape(-1))
    return _attention_g2(gath, hl, att, b)


def kernel(x, params):
    xb = _feature_transform(x, params['ft_W0'], params['ft_b0'],
                            params['ft_W1'], params['ft_b1'])
    y = _gat_layer(xb, params, 'l1', NB)
    outs = [y]
    for i in range(2, len(CH) + 1):
        outs.append(_gat_layer(outs[-1], params, 'l' + str(i), 1))
    out = jnp.concatenate(outs, axis=1)
    return _mlp(out, params['m1_W'], params['m1_b'], params['m2_W'],
                params['m2_b'], params['gl_W'], params['gl_b'], params['geod'])
